# baseline (speedup 1.0000x reference)
import dataclasses

import jax
import jax.numpy as jnp
from jax import lax
from jax.experimental import pallas as pl
from jax.experimental.pallas import tpu as pltpu
from jax.experimental.pallas import tpu_sc as plsc

N = 10000
NP_ = 10240
E = 320000
D = 128
DW = 144
H = 8
DH = 16

NC = 2
NS = 16
NWORK = NC * NS
EPT = E // NWORK
C = 40
NCHUNK = EPT // C
NG = NCHUNK // 2
NPT = NP_ // NS


def _qkv_body(x_ref, w_ref, q_ref, k_ref, v_ref):
    x = x_ref[...]
    w = w_ref[...]
    q_ref[...] = jnp.dot(x, w[:, 0:D], preferred_element_type=jnp.float32)
    k_ref[...] = jnp.dot(x, w[:, D:2 * D], preferred_element_type=jnp.float32)
    z = jnp.zeros((x.shape[0], DW - D), jnp.float32)
    v_ref[...] = jnp.concatenate(
        [jnp.dot(x, w[:, 2 * D:3 * D], preferred_element_type=jnp.float32), z],
        axis=1)


def _qkv(feat, wqkv_t, blk=1024):
    grid = (NP_ // blk,)
    outd = jax.ShapeDtypeStruct((NP_, D), jnp.float32)
    outw = jax.ShapeDtypeStruct((NP_, DW), jnp.float32)
    return pl.pallas_call(
        _qkv_body,
        grid=grid,
        in_specs=[
            pl.BlockSpec((blk, D), lambda i: (i, 0)),
            pl.BlockSpec((D, 3 * D), lambda i: (0, 0)),
        ],
        out_specs=[
            pl.BlockSpec((blk, D), lambda i: (i, 0)),
            pl.BlockSpec((blk, D), lambda i: (i, 0)),
            pl.BlockSpec((blk, DW), lambda i: (i, 0)),
        ],
        out_shape=[outd, outd, outw],
    )(feat, wqkv_t)


def _edge_body(q_hbm, k_hbm, v_hbm, esd_hbm, z144_hbm,
               comb_out, comb_sh, qbuf, kbuf, vbuf, wbuf,
               ib0, ib1, sq, sk, sv, si, ss):
    c = lax.axis_index("c")
    s = lax.axis_index("s")
    w = c * NS + s

    pltpu.sync_copy(z144_hbm, vbuf)
    for j in range(NPT // C):
        pltpu.sync_copy(vbuf, comb_sh.at[pl.ds(s * NPT + j * C, C)])

    lane = lax.iota(jnp.int32, 16)
    head_mask = jnp.where(lane < H, 1.0, 0.0)

    plsc.subcore_barrier()

    pltpu.sync_copy(esd_hbm.at[w, 0], ib0)
    pltpu.async_copy(vbuf, comb_sh.at[ib0.at[1]], ss, add=True)
    pltpu.async_copy(q_hbm.at[ib0.at[0]], qbuf, sq)
    pltpu.async_copy(k_hbm.at[ib0.at[1]], kbuf, sk)

    def _dots_loop():
        @plsc.parallel_loop(0, C, unroll=4)
        def _dots(e):
            wvec = jnp.zeros((16,), jnp.float32)
            for h in range(H):
                prod = qbuf[e, pl.ds(DH * h, DH)] * kbuf[e, pl.ds(DH * h, DH)]
                sm = jnp.sum(prod)
                wvec = jnp.where(lane == h, jnp.full((16,), sm, jnp.float32),
                                 wvec)
            wvec = jnp.minimum(jnp.maximum(wvec, -5.0), 5.0) * 4.0
            wvec = jnp.exp(wvec) * head_mask
            wbuf[e, :] = wvec

    def _apply_loop():
        @plsc.parallel_loop(0, C, unroll=4)
        def _apply(e):
            wvec = wbuf[e, :]
            for h in range(H):
                bc = lax.gather(
                    wvec, jnp.full((16, 1), h, jnp.int32),
                    lax.GatherDimensionNumbers(
                        offset_dims=(), collapsed_slice_dims=(0,),
                        start_index_map=(0,)),
                    slice_sizes=(1,),
                    mode=lax.GatherScatterMode.PROMISE_IN_BOUNDS)
                vbuf[e, pl.ds(DH * h, DH)] = vbuf[e, pl.ds(DH * h, DH)] * bc
            vbuf[e, pl.ds(D, 16)] = wvec

    @pl.loop(0, NG)
    def _g(g):
        for b in (0, 1):
            ib = ib0 if b == 0 else ib1
            ib_n = ib1 if b == 0 else ib0
            ch = 2 * g + b
            nch = lax.rem(ch + 1, NCHUNK)
            pltpu.make_async_copy(vbuf, comb_sh.at[ib_n.at[1]], ss).wait()
            pltpu.async_copy(esd_hbm.at[w, nch], ib_n, si)
            pltpu.async_copy(v_hbm.at[ib.at[0]], vbuf, sv)
            pltpu.make_async_copy(q_hbm.at[ib.at[0]], qbuf, sq).wait()
            pltpu.make_async_copy(k_hbm.at[ib.at[1]], kbuf, sk).wait()
            _dots_loop()
            pltpu.make_async_copy(esd_hbm.at[w, nch], ib_n, si).wait()
            pltpu.async_copy(q_hbm.at[ib_n.at[0]], qbuf, sq)
            pltpu.async_copy(k_hbm.at[ib_n.at[1]], kbuf, sk)
            pltpu.make_async_copy(v_hbm.at[ib.at[0]], vbuf, sv).wait()
            _apply_loop()
            pltpu.async_copy(vbuf, comb_sh.at[ib.at[1]], ss, add=True)

    pltpu.make_async_copy(vbuf, comb_sh.at[ib1.at[1]], ss).wait()
    pltpu.make_async_copy(q_hbm.at[ib0.at[0]], qbuf, sq).wait()
    pltpu.make_async_copy(k_hbm.at[ib0.at[1]], kbuf, sk).wait()

    plsc.subcore_barrier()

    for j in range(NPT // C):
        pltpu.sync_copy(comb_sh.at[pl.ds(s * NPT + j * C, C)], vbuf)
        pltpu.sync_copy(vbuf, comb_out.at[c, pl.ds(s * NPT + j * C, C)])


def _edge_pass(q, k, v, esd, z144):
    mesh = plsc.VectorSubcoreMesh(core_axis_name="c", subcore_axis_name="s")
    f32 = jnp.float32
    cp = pltpu.CompilerParams()
    if "needs_layout_passes" in pltpu.CompilerParams.__dataclass_fields__:
        cp = dataclasses.replace(cp, needs_layout_passes=False)
    if "use_tc_tiling_on_sc" in pltpu.CompilerParams.__dataclass_fields__:
        cp = dataclasses.replace(cp, use_tc_tiling_on_sc=False)
    kern = pl.kernel(
        _edge_body,
        out_type=jax.ShapeDtypeStruct((NC, NP_, DW), f32),
        mesh=mesh,
        scratch_types=[
            pltpu.VMEM_SHARED((NP_, DW), f32),
            pltpu.VMEM((C, D), f32),
            pltpu.VMEM((C, D), f32),
            pltpu.VMEM((C, DW), f32),
            pltpu.VMEM((C, 16), f32),
            pltpu.VMEM((2, C), jnp.int32),
            pltpu.VMEM((2, C), jnp.int32),
            pltpu.SemaphoreType.DMA,
            pltpu.SemaphoreType.DMA,
            pltpu.SemaphoreType.DMA,
            pltpu.SemaphoreType.DMA,
            pltpu.SemaphoreType.DMA,
        ],
        compiler_params=cp,
    )
    return kern(q, k, v, esd, z144)


def _post_body(comb_ref, x_ref, wn_ref, wsk_ref, ga_ref, gb_ref,
               dex_ref, ln1g_ref, ln1b_ref, lnfg_ref, lnfb_ref,
               w1_ref, w2_ref, o_ref):
    comb = comb_ref[0] + comb_ref[1]
    aggu = comb[:, 0:D]
    den8 = comb[:, D:D + H]
    den = jnp.dot(den8, dex_ref[...], preferred_element_type=jnp.float32)
    agg = jnp.where(den > 0.0, aggu / jnp.maximum(den, 1e-30), 0.0)

    rst = jnp.dot(agg, wn_ref[...], preferred_element_type=jnp.float32)
    x = x_ref[...]
    skip = jnp.dot(x, wsk_ref[...], preferred_element_type=jnp.float32)

    gl = (jnp.dot(rst, ga_ref[...], preferred_element_type=jnp.float32)
          + jnp.dot(skip, gb_ref[...], preferred_element_type=jnp.float32))
    gate = jax.nn.sigmoid(gl)
    mix = rst * gate + skip * (1.0 - gate)

    mu = jnp.mean(mix, axis=-1, keepdims=True)
    var = jnp.mean((mix - mu) ** 2, axis=-1, keepdims=True)
    h = (mix - mu) / jnp.sqrt(var + 1e-5) * ln1g_ref[...] + ln1b_ref[...]

    mu2 = jnp.mean(h, axis=-1, keepdims=True)
    var2 = jnp.mean((h - mu2) ** 2, axis=-1, keepdims=True)
    fin = (h - mu2) / jnp.sqrt(var2 + 1e-5) * lnfg_ref[...] + lnfb_ref[...]

    ffn = jnp.dot(
        jnp.maximum(jnp.dot(fin, w1_ref[...],
                            preferred_element_type=jnp.float32), 0.0),
        w2_ref[...], preferred_element_type=jnp.float32)
    o_ref[...] = h + ffn


def _post(comb2, feat, wn_t, wsk_t, ga, gb, dex, ln1g, ln1b, lnfg, lnfb,
          w1_t, w2_t, blk=1024):
    grid = (NP_ // blk,)
    full = lambda shape: pl.BlockSpec(shape, lambda i: tuple(0 for _ in shape))
    return pl.pallas_call(
        _post_body,
        grid=grid,
        in_specs=[
            pl.BlockSpec((NC, blk, DW), lambda i: (0, i, 0)),
            pl.BlockSpec((blk, D), lambda i: (i, 0)),
            full((D, D)),
            full((D, D)),
            full((D, 1)),
            full((D, 1)),
            full((H, D)),
            full((1, D)),
            full((1, D)),
            full((1, D)),
            full((1, D)),
            full((D, D)),
            full((D, D)),
        ],
        out_specs=pl.BlockSpec((blk, D), lambda i: (i, 0)),
        out_shape=jax.ShapeDtypeStruct((NP_, D), jnp.float32),
    )(comb2, feat, wn_t, wsk_t, ga, gb, dex, ln1g, ln1b, lnfg, lnfb,
      w1_t, w2_t)


def kernel(feat, edge_index, Wq, Wk, Wv, Wn, Wskip, Wgres, ln1_g, ln1_b,
           lnf_g, lnf_b, Wffn1, Wffn2):
    esd = jnp.stack([edge_index[0].reshape(NWORK, NCHUNK, C),
                     edge_index[1].reshape(NWORK, NCHUNK, C)], axis=2)
    featp = jnp.pad(feat, ((0, NP_ - N), (0, 0)))

    wqkv_t = jnp.concatenate([Wq.T, Wk.T, Wv.T], axis=1)
    q, k, v = _qkv(featp, wqkv_t)

    z144 = jnp.zeros((C, DW), jnp.float32)
    comb2 = _edge_pass(q, k, v, esd, z144)

    g1 = Wgres[0, 0:D]
    g2 = Wgres[0, D:2 * D]
    g3 = Wgres[0, 2 * D:3 * D]
    ga = (g1 + g3).reshape(D, 1)
    gb = (g2 - g3).reshape(D, 1)
    dex = jnp.repeat(jnp.eye(H, dtype=jnp.float32), DH, axis=1)

    outp = _post(comb2, featp, Wn.T, Wskip.T, ga, gb, dex,
                 ln1_g.reshape(1, D), ln1_b.reshape(1, D),
                 lnf_g.reshape(1, D), lnf_b.reshape(1, D),
                 Wffn1.T, Wffn2.T)
    return outp[:N]


# speedup vs baseline: 1.0622x; 1.0622x over previous
import dataclasses

import jax
import jax.numpy as jnp
from jax import lax
from jax.experimental import pallas as pl
from jax.experimental.pallas import tpu as pltpu
from jax.experimental.pallas import tpu_sc as plsc

N = 10000
NP_ = 10240
E = 320000
D = 128
DW = 144
H = 8
DH = 16

NC = 2
NS = 16
NWORK = NC * NS
EPT = E // NWORK
C = 40
NCHUNK = EPT // C
NG = NCHUNK // 5
NPT = NP_ // NS


def _qkv_body(x_ref, w_ref, q_ref, k_ref, v_ref):
    x = x_ref[...]
    w = w_ref[...]
    q_ref[...] = jnp.dot(x, w[:, 0:D], preferred_element_type=jnp.float32)
    k_ref[...] = jnp.dot(x, w[:, D:2 * D], preferred_element_type=jnp.float32)
    z = jnp.zeros((x.shape[0], DW - D), jnp.float32)
    v_ref[...] = jnp.concatenate(
        [jnp.dot(x, w[:, 2 * D:3 * D], preferred_element_type=jnp.float32), z],
        axis=1)


def _qkv(feat, wqkv_t, blk=1024):
    grid = (NP_ // blk,)
    outd = jax.ShapeDtypeStruct((NP_, D), jnp.float32)
    outw = jax.ShapeDtypeStruct((NP_, DW), jnp.float32)
    return pl.pallas_call(
        _qkv_body,
        grid=grid,
        in_specs=[
            pl.BlockSpec((blk, D), lambda i: (i, 0)),
            pl.BlockSpec((D, 3 * D), lambda i: (0, 0)),
        ],
        out_specs=[
            pl.BlockSpec((blk, D), lambda i: (i, 0)),
            pl.BlockSpec((blk, D), lambda i: (i, 0)),
            pl.BlockSpec((blk, DW), lambda i: (i, 0)),
        ],
        out_shape=[outd, outd, outw],
    )(feat, wqkv_t)


def _edge_body(q_hbm, k_hbm, v_hbm, src_hbm, dst_hbm, z144_hbm,
               comb_out, comb_sh, qbuf, kbuf, vbuf0, vbuf1, wbuf,
               sidxs, didxs, sq, sk, sv, si, ss0, ss1):
    c = lax.axis_index("c")
    s = lax.axis_index("s")
    w = c * NS + s

    pltpu.sync_copy(z144_hbm, vbuf0)
    pltpu.sync_copy(z144_hbm, vbuf1)
    for j in range(NPT // C):
        pltpu.sync_copy(vbuf0, comb_sh.at[pl.ds(s * NPT + j * C, C)])

    lane = lax.iota(jnp.int32, 16)
    head_mask = jnp.where(lane < H, 1.0, 0.0)

    plsc.subcore_barrier()

    base0 = w * EPT
    pltpu.sync_copy(src_hbm.at[pl.ds(base0, C)], sidxs.at[0])
    pltpu.sync_copy(dst_hbm.at[pl.ds(base0, C)], didxs.at[0])
    pltpu.async_copy(vbuf0, comb_sh.at[didxs.at[0]], ss0, add=True)
    pltpu.async_copy(vbuf1, comb_sh.at[didxs.at[0]], ss1, add=True)
    pltpu.async_copy(q_hbm.at[sidxs.at[0]], qbuf, sq)
    pltpu.async_copy(k_hbm.at[didxs.at[0]], kbuf, sk)

    def _dots_loop():
        @plsc.parallel_loop(0, C, unroll=4)
        def _dots(e):
            wvec = jnp.zeros((16,), jnp.float32)
            for h in range(H):
                prod = qbuf[e, pl.ds(DH * h, DH)] * kbuf[e, pl.ds(DH * h, DH)]
                sm = jnp.sum(prod)
                wvec = jnp.where(lane == h, jnp.full((16,), sm, jnp.float32),
                                 wvec)
            wvec = jnp.minimum(jnp.maximum(wvec, -5.0), 5.0) * 4.0
            wvec = jnp.exp(wvec) * head_mask
            wbuf[e, :] = wvec

    def _apply_loop(vbuf):
        @plsc.parallel_loop(0, C, unroll=4)
        def _apply(e):
            wvec = wbuf[e, :]
            for h in range(H):
                bc = lax.gather(
                    wvec, jnp.full((16, 1), h, jnp.int32),
                    lax.GatherDimensionNumbers(
                        offset_dims=(), collapsed_slice_dims=(0,),
                        start_index_map=(0,)),
                    slice_sizes=(1,),
                    mode=lax.GatherScatterMode.PROMISE_IN_BOUNDS)
                vbuf[e, pl.ds(DH * h, DH)] = vbuf[e, pl.ds(DH * h, DH)] * bc
            vbuf[e, pl.ds(D, 16)] = wvec

    @pl.loop(0, NG)
    def _g(g):
        for b in range(5):
            sidx = sidxs.at[b]
            didx = didxs.at[b]
            b1 = (b + 1) % 5
            sidx_n = sidxs.at[b1]
            didx_n = didxs.at[b1]
            ch = 5 * g + b
            vbuf = vbuf0 if b % 2 == 0 else vbuf1
            ssp = ss0 if b % 2 == 0 else ss1
            nbase = w * EPT + lax.rem(ch + 1, NCHUNK) * C
            pltpu.make_async_copy(vbuf, comb_sh.at[didx], ssp).wait()
            pltpu.async_copy(v_hbm.at[sidx], vbuf, sv)
            pltpu.async_copy(src_hbm.at[pl.ds(nbase, C)], sidx_n, si)
            pltpu.async_copy(dst_hbm.at[pl.ds(nbase, C)], didx_n, si)
            pltpu.make_async_copy(q_hbm.at[sidx], qbuf, sq).wait()
            pltpu.make_async_copy(k_hbm.at[didx], kbuf, sk).wait()
            _dots_loop()
            pltpu.make_async_copy(src_hbm.at[pl.ds(nbase, C)], sidx_n,
                                  si).wait()
            pltpu.make_async_copy(dst_hbm.at[pl.ds(nbase, C)], didx_n,
                                  si).wait()
            pltpu.async_copy(q_hbm.at[sidx_n], qbuf, sq)
            pltpu.async_copy(k_hbm.at[didx_n], kbuf, sk)
            pltpu.make_async_copy(v_hbm.at[sidx], vbuf, sv).wait()
            _apply_loop(vbuf)
            pltpu.async_copy(vbuf, comb_sh.at[didx], ssp, add=True)

    pltpu.make_async_copy(vbuf0, comb_sh.at[didxs.at[0]], ss0).wait()
    pltpu.make_async_copy(vbuf1, comb_sh.at[didxs.at[0]], ss1).wait()
    pltpu.make_async_copy(q_hbm.at[sidxs.at[0]], qbuf, sq).wait()
    pltpu.make_async_copy(k_hbm.at[didxs.at[0]], kbuf, sk).wait()

    plsc.subcore_barrier()

    for j in range(NPT // C):
        pltpu.sync_copy(comb_sh.at[pl.ds(s * NPT + j * C, C)], vbuf0)
        pltpu.sync_copy(vbuf0, comb_out.at[c, pl.ds(s * NPT + j * C, C)])


def _edge_pass(q, k, v, src, dst, z144):
    mesh = plsc.VectorSubcoreMesh(core_axis_name="c", subcore_axis_name="s")
    f32 = jnp.float32
    cp = pltpu.CompilerParams()
    if "needs_layout_passes" in pltpu.CompilerParams.__dataclass_fields__:
        cp = dataclasses.replace(cp, needs_layout_passes=False)
    if "use_tc_tiling_on_sc" in pltpu.CompilerParams.__dataclass_fields__:
        cp = dataclasses.replace(cp, use_tc_tiling_on_sc=False)
    kern = pl.kernel(
        _edge_body,
        out_type=jax.ShapeDtypeStruct((NC, NP_, DW), f32),
        mesh=mesh,
        scratch_types=[
            pltpu.VMEM_SHARED((NP_, DW), f32),
            pltpu.VMEM((C, D), f32),
            pltpu.VMEM((C, D), f32),
            pltpu.VMEM((C, DW), f32),
            pltpu.VMEM((C, DW), f32),
            pltpu.VMEM((C, 16), f32),
            pltpu.VMEM((5, C), jnp.int32),
            pltpu.VMEM((5, C), jnp.int32),
            pltpu.SemaphoreType.DMA,
            pltpu.SemaphoreType.DMA,
            pltpu.SemaphoreType.DMA,
            pltpu.SemaphoreType.DMA,
            pltpu.SemaphoreType.DMA,
            pltpu.SemaphoreType.DMA,
        ],
        compiler_params=cp,
    )
    return kern(q, k, v, src, dst, z144)


def _post_body(comb_ref, x_ref, wn_ref, wsk_ref, ga_ref, gb_ref,
               dex_ref, ln1g_ref, ln1b_ref, lnfg_ref, lnfb_ref,
               w1_ref, w2_ref, o_ref):
    comb = comb_ref[0] + comb_ref[1]
    aggu = comb[:, 0:D]
    den8 = comb[:, D:D + H]
    den = jnp.dot(den8, dex_ref[...], preferred_element_type=jnp.float32)
    agg = jnp.where(den > 0.0, aggu / jnp.maximum(den, 1e-30), 0.0)

    rst = jnp.dot(agg, wn_ref[...], preferred_element_type=jnp.float32)
    x = x_ref[...]
    skip = jnp.dot(x, wsk_ref[...], preferred_element_type=jnp.float32)

    gl = (jnp.dot(rst, ga_ref[...], preferred_element_type=jnp.float32)
          + jnp.dot(skip, gb_ref[...], preferred_element_type=jnp.float32))
    gate = jax.nn.sigmoid(gl)
    mix = rst * gate + skip * (1.0 - gate)

    mu = jnp.mean(mix, axis=-1, keepdims=True)
    var = jnp.mean((mix - mu) ** 2, axis=-1, keepdims=True)
    h = (mix - mu) / jnp.sqrt(var + 1e-5) * ln1g_ref[...] + ln1b_ref[...]

    mu2 = jnp.mean(h, axis=-1, keepdims=True)
    var2 = jnp.mean((h - mu2) ** 2, axis=-1, keepdims=True)
    fin = (h - mu2) / jnp.sqrt(var2 + 1e-5) * lnfg_ref[...] + lnfb_ref[...]

    ffn = jnp.dot(
        jnp.maximum(jnp.dot(fin, w1_ref[...],
                            preferred_element_type=jnp.float32), 0.0),
        w2_ref[...], preferred_element_type=jnp.float32)
    o_ref[...] = h + ffn


def _post(comb2, feat, wn_t, wsk_t, ga, gb, dex, ln1g, ln1b, lnfg, lnfb,
          w1_t, w2_t, blk=1024):
    grid = (NP_ // blk,)
    full = lambda shape: pl.BlockSpec(shape, lambda i: tuple(0 for _ in shape))
    return pl.pallas_call(
        _post_body,
        grid=grid,
        in_specs=[
            pl.BlockSpec((NC, blk, DW), lambda i: (0, i, 0)),
            pl.BlockSpec((blk, D), lambda i: (i, 0)),
            full((D, D)),
            full((D, D)),
            full((D, 1)),
            full((D, 1)),
            full((H, D)),
            full((1, D)),
            full((1, D)),
            full((1, D)),
            full((1, D)),
            full((D, D)),
            full((D, D)),
        ],
        out_specs=pl.BlockSpec((blk, D), lambda i: (i, 0)),
        out_shape=jax.ShapeDtypeStruct((NP_, D), jnp.float32),
    )(comb2, feat, wn_t, wsk_t, ga, gb, dex, ln1g, ln1b, lnfg, lnfb,
      w1_t, w2_t)


def kernel(feat, edge_index, Wq, Wk, Wv, Wn, Wskip, Wgres, ln1_g, ln1_b,
           lnf_g, lnf_b, Wffn1, Wffn2):
    src = edge_index[0]
    dst = edge_index[1]
    featp = jnp.pad(feat, ((0, NP_ - N), (0, 0)))

    wqkv_t = jnp.concatenate([Wq.T, Wk.T, Wv.T], axis=1)
    q, k, v = _qkv(featp, wqkv_t)

    z144 = jnp.zeros((C, DW), jnp.float32)
    comb2 = _edge_pass(q, k, v, src, dst, z144)

    g1 = Wgres[0, 0:D]
    g2 = Wgres[0, D:2 * D]
    g3 = Wgres[0, 2 * D:3 * D]
    ga = (g1 + g3).reshape(D, 1)
    gb = (g2 - g3).reshape(D, 1)
    dex = jnp.repeat(jnp.eye(H, dtype=jnp.float32), DH, axis=1)

    outp = _post(comb2, featp, Wn.T, Wskip.T, ga, gb, dex,
                 ln1_g.reshape(1, D), ln1_b.reshape(1, D),
                 lnf_g.reshape(1, D), lnf_b.reshape(1, D),
                 Wffn1.T, Wffn2.T)
    return outp[:N]


# speedup vs baseline: 1.1099x; 1.0449x over previous
import dataclasses

import jax
import jax.numpy as jnp
from jax import lax
from jax.experimental import pallas as pl
from jax.experimental.pallas import tpu as pltpu
from jax.experimental.pallas import tpu_sc as plsc

N = 10000
NP_ = 10240
E = 320000
D = 128
DW = 144
H = 8
DH = 16

NC = 2
NS = 16
NWORK = NC * NS
EPT = E // NWORK
C = 40
NCHUNK = EPT // C
NG = NCHUNK // 2
NPT = NP_ // NS


def _qkv_body(x_ref, w_ref, q_ref, k_ref, v_ref):
    x = x_ref[...]
    w = w_ref[...]
    q_ref[...] = jnp.dot(x, w[:, 0:D], preferred_element_type=jnp.float32)
    k_ref[...] = jnp.dot(x, w[:, D:2 * D], preferred_element_type=jnp.float32)
    z = jnp.zeros((x.shape[0], DW - D), jnp.float32)
    v_ref[...] = jnp.concatenate(
        [jnp.dot(x, w[:, 2 * D:3 * D], preferred_element_type=jnp.float32), z],
        axis=1)


def _qkv(feat, wqkv_t, blk=1024):
    grid = (NP_ // blk,)
    outd = jax.ShapeDtypeStruct((NP_, D), jnp.float32)
    outw = jax.ShapeDtypeStruct((NP_, DW), jnp.float32)
    return pl.pallas_call(
        _qkv_body,
        grid=grid,
        in_specs=[
            pl.BlockSpec((blk, D), lambda i: (i, 0)),
            pl.BlockSpec((D, 3 * D), lambda i: (0, 0)),
        ],
        out_specs=[
            pl.BlockSpec((blk, D), lambda i: (i, 0)),
            pl.BlockSpec((blk, D), lambda i: (i, 0)),
            pl.BlockSpec((blk, DW), lambda i: (i, 0)),
        ],
        out_shape=[outd, outd, outw],
    )(feat, wqkv_t)


def _edge_body(q_hbm, k_hbm, v_hbm, src_hbm, dst_hbm, z144_hbm,
               comb_out, comb_sh, qbuf, kbuf, vbuf, wbuf,
               sidx0, didx0, sidx1, didx1, sq, sk, sv, si, ss):
    c = lax.axis_index("c")
    s = lax.axis_index("s")
    w = c * NS + s

    pltpu.sync_copy(z144_hbm, vbuf)
    for j in range(NPT // C):
        pltpu.sync_copy(vbuf, comb_sh.at[pl.ds(s * NPT + j * C, C)])

    lane = lax.iota(jnp.int32, 16)
    head_mask = jnp.where(lane < H, 1.0, 0.0)

    plsc.subcore_barrier()

    base0 = w * EPT
    pltpu.sync_copy(src_hbm.at[pl.ds(base0, C)], sidx0)
    pltpu.sync_copy(dst_hbm.at[pl.ds(base0, C)], didx0)
    pltpu.async_copy(vbuf, comb_sh.at[didx0], ss, add=True)
    pltpu.async_copy(q_hbm.at[sidx0], qbuf, sq)
    pltpu.async_copy(k_hbm.at[didx0], kbuf, sk)

    def _dots_loop(sidx, didx):
        @plsc.parallel_loop(0, C, unroll=4)
        def _dots(e):
            wvec = jnp.zeros((16,), jnp.float32)
            for h in range(H):
                prod = qbuf[e, pl.ds(DH * h, DH)] * kbuf[e, pl.ds(DH * h, DH)]
                sm = jnp.sum(prod)
                wvec = jnp.where(lane == h, jnp.full((16,), sm, jnp.float32),
                                 wvec)
            wvec = jnp.minimum(jnp.maximum(wvec, -5.0), 5.0) * 4.0
            wvec = jnp.exp(wvec) * head_mask
            wbuf[e, :] = wvec

    def _apply_loop():
        @plsc.parallel_loop(0, C, unroll=4)
        def _apply(e):
            wvec = wbuf[e, :]
            for h in range(H):
                bc = lax.gather(
                    wvec, jnp.full((16, 1), h, jnp.int32),
                    lax.GatherDimensionNumbers(
                        offset_dims=(), collapsed_slice_dims=(0,),
                        start_index_map=(0,)),
                    slice_sizes=(1,),
                    mode=lax.GatherScatterMode.PROMISE_IN_BOUNDS)
                vbuf[e, pl.ds(DH * h, DH)] = vbuf[e, pl.ds(DH * h, DH)] * bc
            vbuf[e, pl.ds(D, 16)] = wvec

    @pl.loop(0, NG)
    def _g(g):
        for b in (0, 1):
            sidx = sidx0 if b == 0 else sidx1
            didx = didx0 if b == 0 else didx1
            sidx_n = sidx1 if b == 0 else sidx0
            didx_n = didx1 if b == 0 else didx0
            ch = 2 * g + b
            nbase = w * EPT + lax.rem(ch + 1, NCHUNK) * C
            pltpu.make_async_copy(vbuf, comb_sh.at[didx_n], ss).wait()
            pltpu.async_copy(src_hbm.at[pl.ds(nbase, C)], sidx_n, si)
            pltpu.async_copy(dst_hbm.at[pl.ds(nbase, C)], didx_n, si)
            pltpu.async_copy(v_hbm.at[sidx], vbuf, sv)
            pltpu.make_async_copy(q_hbm.at[sidx], qbuf, sq).wait()
            pltpu.make_async_copy(k_hbm.at[didx], kbuf, sk).wait()
            _dots_loop(sidx, didx)
            pltpu.make_async_copy(src_hbm.at[pl.ds(nbase, C)], sidx_n,
                                  si).wait()
            pltpu.make_async_copy(dst_hbm.at[pl.ds(nbase, C)], didx_n,
                                  si).wait()
            pltpu.async_copy(q_hbm.at[sidx_n], qbuf, sq)
            pltpu.async_copy(k_hbm.at[didx_n], kbuf, sk)
            pltpu.make_async_copy(v_hbm.at[sidx], vbuf, sv).wait()
            _apply_loop()
            pltpu.async_copy(vbuf, comb_sh.at[didx], ss, add=True)

    pltpu.make_async_copy(vbuf, comb_sh.at[didx1], ss).wait()
    pltpu.make_async_copy(q_hbm.at[sidx0], qbuf, sq).wait()
    pltpu.make_async_copy(k_hbm.at[didx0], kbuf, sk).wait()

    plsc.subcore_barrier()

    for j in range(NPT // C):
        pltpu.sync_copy(comb_sh.at[pl.ds(s * NPT + j * C, C)], vbuf)
        pltpu.sync_copy(vbuf, comb_out.at[c, pl.ds(s * NPT + j * C, C)])


def _edge_pass(q, k, v, src, dst, z144):
    mesh = plsc.VectorSubcoreMesh(core_axis_name="c", subcore_axis_name="s")
    f32 = jnp.float32
    cp = pltpu.CompilerParams()
    if "needs_layout_passes" in pltpu.CompilerParams.__dataclass_fields__:
        cp = dataclasses.replace(cp, needs_layout_passes=False)
    if "use_tc_tiling_on_sc" in pltpu.CompilerParams.__dataclass_fields__:
        cp = dataclasses.replace(cp, use_tc_tiling_on_sc=False)
    kern = pl.kernel(
        _edge_body,
        out_type=jax.ShapeDtypeStruct((NC, NP_, DW), f32),
        mesh=mesh,
        scratch_types=[
            pltpu.VMEM_SHARED((NP_, DW), f32),
            pltpu.VMEM((C, D), f32),
            pltpu.VMEM((C, D), f32),
            pltpu.VMEM((C, DW), f32),
            pltpu.VMEM((C, 16), f32),
            pltpu.VMEM((C,), jnp.int32),
            pltpu.VMEM((C,), jnp.int32),
            pltpu.VMEM((C,), jnp.int32),
            pltpu.VMEM((C,), jnp.int32),
            pltpu.SemaphoreType.DMA,
            pltpu.SemaphoreType.DMA,
            pltpu.SemaphoreType.DMA,
            pltpu.SemaphoreType.DMA,
            pltpu.SemaphoreType.DMA,
        ],
        compiler_params=cp,
    )
    return kern(q, k, v, src, dst, z144)


def _post_body(comb_ref, x_ref, wn_ref, wsk_ref, ga_ref, gb_ref,
               dex_ref, ln1g_ref, ln1b_ref, lnfg_ref, lnfb_ref,
               w1_ref, w2_ref, o_ref):
    comb = comb_ref[0] + comb_ref[1]
    aggu = comb[:, 0:D]
    den8 = comb[:, D:D + H]
    den = jnp.dot(den8, dex_ref[...], preferred_element_type=jnp.float32)
    agg = jnp.where(den > 0.0, aggu / jnp.maximum(den, 1e-30), 0.0)

    rst = jnp.dot(agg, wn_ref[...], preferred_element_type=jnp.float32)
    x = x_ref[...]
    skip = jnp.dot(x, wsk_ref[...], preferred_element_type=jnp.float32)

    gl = (jnp.dot(rst, ga_ref[...], preferred_element_type=jnp.float32)
          + jnp.dot(skip, gb_ref[...], preferred_element_type=jnp.float32))
    gate = jax.nn.sigmoid(gl)
    mix = rst * gate + skip * (1.0 - gate)

    mu = jnp.mean(mix, axis=-1, keepdims=True)
    var = jnp.mean((mix - mu) ** 2, axis=-1, keepdims=True)
    h = (mix - mu) / jnp.sqrt(var + 1e-5) * ln1g_ref[...] + ln1b_ref[...]

    mu2 = jnp.mean(h, axis=-1, keepdims=True)
    var2 = jnp.mean((h - mu2) ** 2, axis=-1, keepdims=True)
    fin = (h - mu2) / jnp.sqrt(var2 + 1e-5) * lnfg_ref[...] + lnfb_ref[...]

    ffn = jnp.dot(
        jnp.maximum(jnp.dot(fin, w1_ref[...],
                            preferred_element_type=jnp.float32), 0.0),
        w2_ref[...], preferred_element_type=jnp.float32)
    o_ref[...] = h + ffn


def _post(comb2, feat, wn_t, wsk_t, ga, gb, dex, ln1g, ln1b, lnfg, lnfb,
          w1_t, w2_t, blk=1024):
    grid = (NP_ // blk,)
    full = lambda shape: pl.BlockSpec(shape, lambda i: tuple(0 for _ in shape))
    return pl.pallas_call(
        _post_body,
        grid=grid,
        in_specs=[
            pl.BlockSpec((NC, blk, DW), lambda i: (0, i, 0)),
            pl.BlockSpec((blk, D), lambda i: (i, 0)),
            full((D, D)),
            full((D, D)),
            full((D, 1)),
            full((D, 1)),
            full((H, D)),
            full((1, D)),
            full((1, D)),
            full((1, D)),
            full((1, D)),
            full((D, D)),
            full((D, D)),
        ],
        out_specs=pl.BlockSpec((blk, D), lambda i: (i, 0)),
        out_shape=jax.ShapeDtypeStruct((NP_, D), jnp.float32),
    )(comb2, feat, wn_t, wsk_t, ga, gb, dex, ln1g, ln1b, lnfg, lnfb,
      w1_t, w2_t)


def kernel(feat, edge_index, Wq, Wk, Wv, Wn, Wskip, Wgres, ln1_g, ln1_b,
           lnf_g, lnf_b, Wffn1, Wffn2):
    src = edge_index[0]
    dst = edge_index[1]
    featp = jnp.pad(feat, ((0, NP_ - N), (0, 0)))

    wqkv_t = jnp.concatenate([Wq.T, Wk.T, Wv.T], axis=1)
    q, k, v = _qkv(featp, wqkv_t)

    z144 = jnp.zeros((C, DW), jnp.float32)
    comb2 = _edge_pass(q, k, v, src, dst, z144)

    g1 = Wgres[0, 0:D]
    g2 = Wgres[0, D:2 * D]
    g3 = Wgres[0, 2 * D:3 * D]
    ga = (g1 + g3).reshape(D, 1)
    gb = (g2 - g3).reshape(D, 1)
    dex = jnp.repeat(jnp.eye(H, dtype=jnp.float32), DH, axis=1)

    outp = _post(comb2, featp, Wn.T, Wskip.T, ga, gb, dex,
                 ln1_g.reshape(1, D), ln1_b.reshape(1, D),
                 lnf_g.reshape(1, D), lnf_b.reshape(1, D),
                 Wffn1.T, Wffn2.T)
    return outp[:N]
